# baseline (speedup 1.0000x reference)
import functools

import jax
import jax.numpy as jnp
from jax import lax
from jax.experimental import pallas as pl
from jax.experimental.pallas import tpu as pltpu
from jax.experimental.pallas import tpu_sc as plsc

N_NODES = 10000
N_EDGES = 320000
D = 128

NC = 2
NS = 16
NW = NC * NS

CHUNK = 128
NCH = 80
E_PAD = NW * CHUNK * NCH
NCH0 = 104
NCH1 = 56
NCHMAX = max(NCH0, NCH1)
SEGLEN = 8
NBUF = 2
N_ACC = 10240
TRASH = N_NODES
ROWS_PER_TILE = N_ACC // NS

_mesh = plsc.VectorSubcoreMesh(core_axis_name="c", subcore_axis_name="s",
                               num_cores=NC, num_subcores=NS)


def _deg_body(dst_hbm, onehot_hbm, zeros_hbm, out_hbm, dst_v, ones_v, acc):
    c = lax.axis_index("c")
    s = lax.axis_index("s")
    w = c * NS + s
    pltpu.sync_copy(dst_hbm.at[w], dst_v)
    pltpu.sync_copy(onehot_hbm, ones_v)
    r0 = s * ROWS_PER_TILE
    pltpu.sync_copy(zeros_hbm.at[pl.ds(r0, ROWS_PER_TILE)],
                    acc.at[pl.ds(r0, ROWS_PER_TILE)])
    plsc.subcore_barrier()

    def step(j, carry):
        pltpu.sync_copy(ones_v, acc.at[dst_v.at[j]], add=True)
        return carry

    lax.fori_loop(0, NCH, step, 0, unroll=False)
    plsc.subcore_barrier()
    pltpu.sync_copy(acc.at[pl.ds(r0, ROWS_PER_TILE)],
                    out_hbm.at[c, pl.ds(r0, ROWS_PER_TILE)])


def _scatter_body(src_hbm, dst_hbm, lins_hbm, zeros_hbm, out_hbm,
                  src_v, dst_v, b0, b1, acc, sem):
    c = lax.axis_index("c")
    s = lax.axis_index("s")
    w = c * NS + s
    n_seg = jnp.where(c == 0, NCH0 // SEGLEN, NCH1 // SEGLEN)
    r0 = s * ROWS_PER_TILE
    pltpu.sync_copy(zeros_hbm.at[pl.ds(r0, ROWS_PER_TILE)],
                    acc.at[pl.ds(r0, ROWS_PER_TILE)])
    plsc.subcore_barrier()
    bufs = [b0, b1]

    def seg_body(sg, carry):
        pltpu.sync_copy(src_hbm.at[w, pl.ds(sg * SEGLEN, SEGLEN)], src_v)
        pltpu.sync_copy(dst_hbm.at[w, pl.ds(sg * SEGLEN, SEGLEN)], dst_v)

        def grp(t, carry2):
            descs = []
            for b in range(NBUF):
                descs.append(pltpu.async_copy(
                    lins_hbm.at[src_v.at[NBUF * t + b]], bufs[b], sem))
            for d_ in descs:
                d_.wait()
            for b in range(NBUF):
                pltpu.sync_copy(bufs[b], acc.at[dst_v.at[NBUF * t + b]],
                                add=True)
            return carry2

        lax.fori_loop(0, SEGLEN // NBUF, grp, 0, unroll=False)
        return carry

    lax.fori_loop(0, n_seg, seg_body, 0, unroll=False)
    plsc.subcore_barrier()
    pltpu.sync_copy(acc.at[pl.ds(r0, ROWS_PER_TILE)],
                    out_hbm.at[c, pl.ds(r0, ROWS_PER_TILE)])


def _make_sc_kernels(interpret=False):
    deg = pl.kernel(
        _deg_body,
        out_type=jax.ShapeDtypeStruct((NC, N_ACC, 128), jnp.float32),
        mesh=_mesh,
        scratch_types=[
            pltpu.VMEM((NCH, CHUNK), jnp.int32),
            pltpu.VMEM((CHUNK, 128), jnp.float32),
            pltpu.VMEM_SHARED((N_ACC, 128), jnp.float32),
        ],
        interpret=interpret,
    )
    scat = pl.kernel(
        _scatter_body,
        out_type=jax.ShapeDtypeStruct((NC, N_ACC, D), jnp.float32),
        mesh=_mesh,
        scratch_types=(
            [pltpu.VMEM((SEGLEN, CHUNK), jnp.int32)] * 2
            + [pltpu.VMEM((CHUNK, D), jnp.float32)] * NBUF
            + [pltpu.VMEM_SHARED((N_ACC, D), jnp.float32)]
            + [pltpu.SemaphoreType.DMA]
        ),
        interpret=interpret,
    )
    return deg, scat


_deg_kernel, _scatter_kernel = _make_sc_kernels()


def _lins_body(x_ref, w_ref, d0_ref, d1_ref, o_ref):
    deg = d0_ref[:, 0:1] + d1_ref[:, 0:1] + 1.0
    dinv = lax.rsqrt(deg)
    lin = jnp.dot(x_ref[...], w_ref[...], preferred_element_type=jnp.float32)
    o_ref[...] = lin * dinv


def _lins_call(x, W, d0, d1):
    nb = 10
    rb = N_NODES // nb
    return pl.pallas_call(
        _lins_body,
        grid=(nb,),
        in_specs=[
            pl.BlockSpec((rb, D), lambda i: (i, 0)),
            pl.BlockSpec((D, D), lambda i: (0, 0)),
            pl.BlockSpec((rb, D), lambda i: (i, 0)),
            pl.BlockSpec((rb, D), lambda i: (i, 0)),
        ],
        out_specs=pl.BlockSpec((rb, D), lambda i: (i, 0)),
        out_shape=jax.ShapeDtypeStruct((N_NODES, D), jnp.float32),
    )(x, W, d0, d1)


def _out_body(p0_ref, p1_ref, lins_ref, d0_ref, d1_ref, b_ref, o_ref):
    deg = d0_ref[:, 0:1] + d1_ref[:, 0:1] + 1.0
    dinv = lax.rsqrt(deg)
    tot = (p0_ref[...] + p1_ref[...] + lins_ref[...]) * dinv + b_ref[...]
    o_ref[...] = jnp.maximum(tot, 0.0)


def _out_call(p0, p1, lins, d0, d1, b):
    nb = 10
    rb = N_NODES // nb
    return pl.pallas_call(
        _out_body,
        grid=(nb,),
        in_specs=[
            pl.BlockSpec((rb, D), lambda i: (i, 0)),
            pl.BlockSpec((rb, D), lambda i: (i, 0)),
            pl.BlockSpec((rb, D), lambda i: (i, 0)),
            pl.BlockSpec((rb, D), lambda i: (i, 0)),
            pl.BlockSpec((rb, D), lambda i: (i, 0)),
            pl.BlockSpec((1, D), lambda i: (0, 0)),
        ],
        out_specs=pl.BlockSpec((rb, D), lambda i: (i, 0)),
        out_shape=jax.ShapeDtypeStruct((N_NODES, D), jnp.float32),
    )(p0, p1, lins, d0, d1, b)


@jax.jit
def kernel(x, edge_index, W, b):
    src = edge_index[0].astype(jnp.int32)
    dst = edge_index[1].astype(jnp.int32)
    pad = E_PAD - N_EDGES
    src_p = jnp.concatenate([src, jnp.zeros((pad,), jnp.int32)])
    dst_p = jnp.concatenate([dst, jnp.full((pad,), TRASH, jnp.int32)])
    src_t = src_p.reshape(NW, NCH, CHUNK)
    dst_t = dst_p.reshape(NW, NCH, CHUNK)

    n0 = NS * NCH0 * CHUNK
    def _split(flat, fill):
        a0 = flat[:n0].reshape(NS, NCH0, CHUNK)
        a1 = flat[n0:].reshape(NS, NCH1, CHUNK)
        a0p = jnp.pad(a0, ((0, 0), (0, NCHMAX - NCH0), (0, 0)),
                      constant_values=fill)
        a1p = jnp.pad(a1, ((0, 0), (0, NCHMAX - NCH1), (0, 0)),
                      constant_values=fill)
        return jnp.concatenate([a0p, a1p], axis=0)
    src_u = _split(src_p, 0)
    dst_u = _split(dst_p, TRASH)

    onehot = jnp.zeros((CHUNK, 128), jnp.float32).at[:, 0].set(1.0)
    zeros16 = jnp.zeros((N_ACC, 128), jnp.float32)
    zerosD = jnp.zeros((N_ACC, D), jnp.float32)

    degacc = _deg_kernel(dst_t, onehot, zeros16)
    d0 = degacc[0]
    d1 = degacc[1]

    lins = _lins_call(x, W, d0[:N_NODES], d1[:N_NODES])

    parts = _scatter_kernel(src_u, dst_u, lins, zerosD)

    return _out_call(parts[0, :N_NODES], parts[1, :N_NODES], lins,
                     d0[:N_NODES], d1[:N_NODES], b.reshape(1, D))


# speedup vs baseline: 1.0863x; 1.0863x over previous
import functools

import jax
import jax.numpy as jnp
from jax import lax
from jax.experimental import pallas as pl
from jax.experimental.pallas import tpu as pltpu
from jax.experimental.pallas import tpu_sc as plsc

N_NODES = 10000
N_EDGES = 320000
D = 128

NC = 2
NS = 16
NW = NC * NS

CHUNK = 128
NCH = 80
E_PAD = NW * CHUNK * NCH
NCH0 = 128
NCH1 = 32
NCHMAX = max(NCH0, NCH1)
SEGLEN = 8
NBUF = 2
N_ACC = 10240
TRASH = N_NODES
ROWS_PER_TILE = N_ACC // NS

_mesh = plsc.VectorSubcoreMesh(core_axis_name="c", subcore_axis_name="s",
                               num_cores=NC, num_subcores=NS)


def _deg_body(dst_hbm, onehot_hbm, zeros_hbm, out_hbm, dst_v, ones_v, acc):
    c = lax.axis_index("c")
    s = lax.axis_index("s")
    w = c * NS + s
    pltpu.sync_copy(dst_hbm.at[w], dst_v)
    pltpu.sync_copy(onehot_hbm, ones_v)
    r0 = s * ROWS_PER_TILE
    pltpu.sync_copy(zeros_hbm.at[pl.ds(r0, ROWS_PER_TILE)],
                    acc.at[pl.ds(r0, ROWS_PER_TILE)])
    plsc.subcore_barrier()

    def step(j, carry):
        pltpu.sync_copy(ones_v, acc.at[dst_v.at[j]], add=True)
        return carry

    lax.fori_loop(0, NCH, step, 0, unroll=False)
    plsc.subcore_barrier()
    pltpu.sync_copy(acc.at[pl.ds(r0, ROWS_PER_TILE)],
                    out_hbm.at[c, pl.ds(r0, ROWS_PER_TILE)])


def _scatter_body(src_hbm, dst_hbm, lins_hbm, zeros_hbm, out_hbm,
                  src_v, dst_v, b0, b1, acc, sem):
    c = lax.axis_index("c")
    s = lax.axis_index("s")
    w = c * NS + s
    n_seg = jnp.where(c == 0, NCH0 // SEGLEN, NCH1 // SEGLEN)
    r0 = s * ROWS_PER_TILE
    pltpu.sync_copy(zeros_hbm.at[pl.ds(r0, ROWS_PER_TILE)],
                    acc.at[pl.ds(r0, ROWS_PER_TILE)])
    plsc.subcore_barrier()
    bufs = [b0, b1]

    def seg_body(sg, carry):
        pltpu.sync_copy(src_hbm.at[w, pl.ds(sg * SEGLEN, SEGLEN)], src_v)
        pltpu.sync_copy(dst_hbm.at[w, pl.ds(sg * SEGLEN, SEGLEN)], dst_v)

        def grp(t, carry2):
            descs = []
            for b in range(NBUF):
                descs.append(pltpu.async_copy(
                    lins_hbm.at[src_v.at[NBUF * t + b]], bufs[b], sem))
            for d_ in descs:
                d_.wait()
            for b in range(NBUF):
                pltpu.sync_copy(bufs[b], acc.at[dst_v.at[NBUF * t + b]],
                                add=True)
            return carry2

        lax.fori_loop(0, SEGLEN // NBUF, grp, 0, unroll=False)
        return carry

    lax.fori_loop(0, n_seg, seg_body, 0, unroll=False)
    plsc.subcore_barrier()
    pltpu.sync_copy(acc.at[pl.ds(r0, ROWS_PER_TILE)],
                    out_hbm.at[c, pl.ds(r0, ROWS_PER_TILE)])


def _make_sc_kernels(interpret=False):
    deg = pl.kernel(
        _deg_body,
        out_type=jax.ShapeDtypeStruct((NC, N_ACC, 128), jnp.float32),
        mesh=_mesh,
        scratch_types=[
            pltpu.VMEM((NCH, CHUNK), jnp.int32),
            pltpu.VMEM((CHUNK, 128), jnp.float32),
            pltpu.VMEM_SHARED((N_ACC, 128), jnp.float32),
        ],
        interpret=interpret,
    )
    scat = pl.kernel(
        _scatter_body,
        out_type=jax.ShapeDtypeStruct((NC, N_ACC, D), jnp.float32),
        mesh=_mesh,
        scratch_types=(
            [pltpu.VMEM((SEGLEN, CHUNK), jnp.int32)] * 2
            + [pltpu.VMEM((CHUNK, D), jnp.float32)] * NBUF
            + [pltpu.VMEM_SHARED((N_ACC, D), jnp.float32)]
            + [pltpu.SemaphoreType.DMA]
        ),
        interpret=interpret,
    )
    return deg, scat


_deg_kernel, _scatter_kernel = _make_sc_kernels()


def _lins_body(x_ref, w_ref, d0_ref, d1_ref, o_ref):
    deg = d0_ref[:, 0:1] + d1_ref[:, 0:1] + 1.0
    dinv = lax.rsqrt(deg)
    lin = jnp.dot(x_ref[...], w_ref[...], preferred_element_type=jnp.float32)
    o_ref[...] = lin * dinv


def _lins_call(x, W, d0, d1):
    nb = 10
    rb = N_NODES // nb
    return pl.pallas_call(
        _lins_body,
        grid=(nb,),
        in_specs=[
            pl.BlockSpec((rb, D), lambda i: (i, 0)),
            pl.BlockSpec((D, D), lambda i: (0, 0)),
            pl.BlockSpec((rb, D), lambda i: (i, 0)),
            pl.BlockSpec((rb, D), lambda i: (i, 0)),
        ],
        out_specs=pl.BlockSpec((rb, D), lambda i: (i, 0)),
        out_shape=jax.ShapeDtypeStruct((N_NODES, D), jnp.float32),
    )(x, W, d0, d1)


def _out_body(p0_ref, p1_ref, lins_ref, d0_ref, d1_ref, b_ref, o_ref):
    deg = d0_ref[:, 0:1] + d1_ref[:, 0:1] + 1.0
    dinv = lax.rsqrt(deg)
    tot = (p0_ref[...] + p1_ref[...] + lins_ref[...]) * dinv + b_ref[...]
    o_ref[...] = jnp.maximum(tot, 0.0)


def _out_call(p0, p1, lins, d0, d1, b):
    nb = 10
    rb = N_NODES // nb
    return pl.pallas_call(
        _out_body,
        grid=(nb,),
        in_specs=[
            pl.BlockSpec((rb, D), lambda i: (i, 0)),
            pl.BlockSpec((rb, D), lambda i: (i, 0)),
            pl.BlockSpec((rb, D), lambda i: (i, 0)),
            pl.BlockSpec((rb, D), lambda i: (i, 0)),
            pl.BlockSpec((rb, D), lambda i: (i, 0)),
            pl.BlockSpec((1, D), lambda i: (0, 0)),
        ],
        out_specs=pl.BlockSpec((rb, D), lambda i: (i, 0)),
        out_shape=jax.ShapeDtypeStruct((N_NODES, D), jnp.float32),
    )(p0, p1, lins, d0, d1, b)


@jax.jit
def kernel(x, edge_index, W, b):
    src = edge_index[0].astype(jnp.int32)
    dst = edge_index[1].astype(jnp.int32)
    pad = E_PAD - N_EDGES
    src_p = jnp.concatenate([src, jnp.zeros((pad,), jnp.int32)])
    dst_p = jnp.concatenate([dst, jnp.full((pad,), TRASH, jnp.int32)])
    src_t = src_p.reshape(NW, NCH, CHUNK)
    dst_t = dst_p.reshape(NW, NCH, CHUNK)

    n0 = NS * NCH0 * CHUNK
    def _split(flat, fill):
        a0 = flat[:n0].reshape(NS, NCH0, CHUNK)
        a1 = flat[n0:].reshape(NS, NCH1, CHUNK)
        a0p = jnp.pad(a0, ((0, 0), (0, NCHMAX - NCH0), (0, 0)),
                      constant_values=fill)
        a1p = jnp.pad(a1, ((0, 0), (0, NCHMAX - NCH1), (0, 0)),
                      constant_values=fill)
        return jnp.concatenate([a0p, a1p], axis=0)
    src_u = _split(src_p, 0)
    dst_u = _split(dst_p, TRASH)

    onehot = jnp.zeros((CHUNK, 128), jnp.float32).at[:, 0].set(1.0)
    zeros16 = jnp.zeros((N_ACC, 128), jnp.float32)
    zerosD = jnp.zeros((N_ACC, D), jnp.float32)

    degacc = _deg_kernel(dst_t, onehot, zeros16)
    d0 = degacc[0]
    d1 = degacc[1]

    lins = _lins_call(x, W, d0[:N_NODES], d1[:N_NODES])

    parts = _scatter_kernel(src_u, dst_u, lins, zerosD)

    return _out_call(parts[0, :N_NODES], parts[1, :N_NODES], lins,
                     d0[:N_NODES], d1[:N_NODES], b.reshape(1, D))


# speedup vs baseline: 1.1029x; 1.0152x over previous
import functools

import jax
import jax.numpy as jnp
from jax import lax
from jax.experimental import pallas as pl
from jax.experimental.pallas import tpu as pltpu
from jax.experimental.pallas import tpu_sc as plsc

N_NODES = 10000
N_EDGES = 320000
D = 128

NC = 2
NS = 16
NW = NC * NS

CHUNK = 128
NCH = 80
E_PAD = NW * CHUNK * NCH
NCH0 = 136
NCH1 = 24
NCHMAX = max(NCH0, NCH1)
SEGLEN = 8
NBUF = 2
N_ACC = 10240
TRASH = N_NODES
ROWS_PER_TILE = N_ACC // NS

_mesh = plsc.VectorSubcoreMesh(core_axis_name="c", subcore_axis_name="s",
                               num_cores=NC, num_subcores=NS)


def _deg_body(dst_hbm, onehot_hbm, zeros_hbm, out_hbm, dst_v, ones_v, acc):
    c = lax.axis_index("c")
    s = lax.axis_index("s")
    w = c * NS + s
    pltpu.sync_copy(dst_hbm.at[w], dst_v)
    pltpu.sync_copy(onehot_hbm, ones_v)
    r0 = s * ROWS_PER_TILE
    pltpu.sync_copy(zeros_hbm.at[pl.ds(r0, ROWS_PER_TILE)],
                    acc.at[pl.ds(r0, ROWS_PER_TILE)])
    plsc.subcore_barrier()

    def step(j, carry):
        pltpu.sync_copy(ones_v, acc.at[dst_v.at[j]], add=True)
        return carry

    lax.fori_loop(0, NCH, step, 0, unroll=False)
    plsc.subcore_barrier()
    pltpu.sync_copy(acc.at[pl.ds(r0, ROWS_PER_TILE)],
                    out_hbm.at[c, pl.ds(r0, ROWS_PER_TILE)])


def _scatter_body(src_hbm, dst_hbm, lins_hbm, zeros_hbm, out_hbm,
                  src_v, dst_v, b0, b1, acc, sem):
    c = lax.axis_index("c")
    s = lax.axis_index("s")
    w = c * NS + s
    n_seg = jnp.where(c == 0, NCH0 // SEGLEN, NCH1 // SEGLEN)
    r0 = s * ROWS_PER_TILE
    pltpu.sync_copy(zeros_hbm.at[pl.ds(r0, ROWS_PER_TILE)],
                    acc.at[pl.ds(r0, ROWS_PER_TILE)])
    plsc.subcore_barrier()
    bufs = [b0, b1]

    def seg_body(sg, carry):
        pltpu.sync_copy(src_hbm.at[w, pl.ds(sg * SEGLEN, SEGLEN)], src_v)
        pltpu.sync_copy(dst_hbm.at[w, pl.ds(sg * SEGLEN, SEGLEN)], dst_v)

        def grp(t, carry2):
            descs = []
            for b in range(NBUF):
                descs.append(pltpu.async_copy(
                    lins_hbm.at[src_v.at[NBUF * t + b]], bufs[b], sem))
            for d_ in descs:
                d_.wait()
            for b in range(NBUF):
                pltpu.sync_copy(bufs[b], acc.at[dst_v.at[NBUF * t + b]],
                                add=True)
            return carry2

        lax.fori_loop(0, SEGLEN // NBUF, grp, 0, unroll=False)
        return carry

    lax.fori_loop(0, n_seg, seg_body, 0, unroll=False)
    plsc.subcore_barrier()
    pltpu.sync_copy(acc.at[pl.ds(r0, ROWS_PER_TILE)],
                    out_hbm.at[c, pl.ds(r0, ROWS_PER_TILE)])


def _make_sc_kernels(interpret=False):
    deg = pl.kernel(
        _deg_body,
        out_type=jax.ShapeDtypeStruct((NC, N_ACC, 128), jnp.float32),
        mesh=_mesh,
        scratch_types=[
            pltpu.VMEM((NCH, CHUNK), jnp.int32),
            pltpu.VMEM((CHUNK, 128), jnp.float32),
            pltpu.VMEM_SHARED((N_ACC, 128), jnp.float32),
        ],
        interpret=interpret,
    )
    scat = pl.kernel(
        _scatter_body,
        out_type=jax.ShapeDtypeStruct((NC, N_ACC, D), jnp.float32),
        mesh=_mesh,
        scratch_types=(
            [pltpu.VMEM((SEGLEN, CHUNK), jnp.int32)] * 2
            + [pltpu.VMEM((CHUNK, D), jnp.float32)] * NBUF
            + [pltpu.VMEM_SHARED((N_ACC, D), jnp.float32)]
            + [pltpu.SemaphoreType.DMA]
        ),
        interpret=interpret,
    )
    return deg, scat


_deg_kernel, _scatter_kernel = _make_sc_kernels()


def _lins_body(x_ref, w_ref, d0_ref, d1_ref, o_ref):
    deg = d0_ref[:, 0:1] + d1_ref[:, 0:1] + 1.0
    dinv = lax.rsqrt(deg)
    lin = jnp.dot(x_ref[...], w_ref[...], preferred_element_type=jnp.float32)
    o_ref[...] = lin * dinv


def _lins_call(x, W, d0, d1):
    nb = 10
    rb = N_NODES // nb
    return pl.pallas_call(
        _lins_body,
        grid=(nb,),
        in_specs=[
            pl.BlockSpec((rb, D), lambda i: (i, 0)),
            pl.BlockSpec((D, D), lambda i: (0, 0)),
            pl.BlockSpec((rb, D), lambda i: (i, 0)),
            pl.BlockSpec((rb, D), lambda i: (i, 0)),
        ],
        out_specs=pl.BlockSpec((rb, D), lambda i: (i, 0)),
        out_shape=jax.ShapeDtypeStruct((N_NODES, D), jnp.float32),
    )(x, W, d0, d1)


def _out_body(p0_ref, p1_ref, lins_ref, d0_ref, d1_ref, b_ref, o_ref):
    deg = d0_ref[:, 0:1] + d1_ref[:, 0:1] + 1.0
    dinv = lax.rsqrt(deg)
    tot = (p0_ref[...] + p1_ref[...] + lins_ref[...]) * dinv + b_ref[...]
    o_ref[...] = jnp.maximum(tot, 0.0)


def _out_call(p0, p1, lins, d0, d1, b):
    nb = 10
    rb = N_NODES // nb
    return pl.pallas_call(
        _out_body,
        grid=(nb,),
        in_specs=[
            pl.BlockSpec((rb, D), lambda i: (i, 0)),
            pl.BlockSpec((rb, D), lambda i: (i, 0)),
            pl.BlockSpec((rb, D), lambda i: (i, 0)),
            pl.BlockSpec((rb, D), lambda i: (i, 0)),
            pl.BlockSpec((rb, D), lambda i: (i, 0)),
            pl.BlockSpec((1, D), lambda i: (0, 0)),
        ],
        out_specs=pl.BlockSpec((rb, D), lambda i: (i, 0)),
        out_shape=jax.ShapeDtypeStruct((N_NODES, D), jnp.float32),
    )(p0, p1, lins, d0, d1, b)


@jax.jit
def kernel(x, edge_index, W, b):
    src = edge_index[0].astype(jnp.int32)
    dst = edge_index[1].astype(jnp.int32)
    pad = E_PAD - N_EDGES
    src_p = jnp.concatenate([src, jnp.zeros((pad,), jnp.int32)])
    dst_p = jnp.concatenate([dst, jnp.full((pad,), TRASH, jnp.int32)])
    src_t = src_p.reshape(NW, NCH, CHUNK)
    dst_t = dst_p.reshape(NW, NCH, CHUNK)

    n0 = NS * NCH0 * CHUNK
    def _split(flat, fill):
        a0 = flat[:n0].reshape(NS, NCH0, CHUNK)
        a1 = flat[n0:].reshape(NS, NCH1, CHUNK)
        a0p = jnp.pad(a0, ((0, 0), (0, NCHMAX - NCH0), (0, 0)),
                      constant_values=fill)
        a1p = jnp.pad(a1, ((0, 0), (0, NCHMAX - NCH1), (0, 0)),
                      constant_values=fill)
        return jnp.concatenate([a0p, a1p], axis=0)
    src_u = _split(src_p, 0)
    dst_u = _split(dst_p, TRASH)

    onehot = jnp.zeros((CHUNK, 128), jnp.float32).at[:, 0].set(1.0)
    zeros16 = jnp.zeros((N_ACC, 128), jnp.float32)
    zerosD = jnp.zeros((N_ACC, D), jnp.float32)

    degacc = _deg_kernel(dst_t, onehot, zeros16)
    d0 = degacc[0]
    d1 = degacc[1]

    lins = _lins_call(x, W, d0[:N_NODES], d1[:N_NODES])

    parts = _scatter_kernel(src_u, dst_u, lins, zerosD)

    return _out_call(parts[0, :N_NODES], parts[1, :N_NODES], lins,
                     d0[:N_NODES], d1[:N_NODES], b.reshape(1, D))


# speedup vs baseline: 1.1251x; 1.0202x over previous
import functools

import jax
import jax.numpy as jnp
from jax import lax
from jax.experimental import pallas as pl
from jax.experimental.pallas import tpu as pltpu
from jax.experimental.pallas import tpu_sc as plsc

N_NODES = 10000
N_EDGES = 320000
D = 128

NC = 2
NS = 16
NW = NC * NS

CHUNK = 128
NCH = 80
E_PAD = NW * CHUNK * NCH
NCH0 = 144
NCH1 = 16
NCHMAX = max(NCH0, NCH1)
SEGLEN = 8
NBUF = 2
N_ACC = 10240
TRASH = N_NODES
ROWS_PER_TILE = N_ACC // NS

_mesh = plsc.VectorSubcoreMesh(core_axis_name="c", subcore_axis_name="s",
                               num_cores=NC, num_subcores=NS)


def _deg_body(dst_hbm, onehot_hbm, zeros_hbm, out_hbm, dst_v, ones_v, acc):
    c = lax.axis_index("c")
    s = lax.axis_index("s")
    w = c * NS + s
    pltpu.sync_copy(dst_hbm.at[w], dst_v)
    pltpu.sync_copy(onehot_hbm, ones_v)
    r0 = s * ROWS_PER_TILE
    pltpu.sync_copy(zeros_hbm.at[pl.ds(r0, ROWS_PER_TILE)],
                    acc.at[pl.ds(r0, ROWS_PER_TILE)])
    plsc.subcore_barrier()

    def step(j, carry):
        pltpu.sync_copy(ones_v, acc.at[dst_v.at[j]], add=True)
        return carry

    lax.fori_loop(0, NCH, step, 0, unroll=False)
    plsc.subcore_barrier()
    pltpu.sync_copy(acc.at[pl.ds(r0, ROWS_PER_TILE)],
                    out_hbm.at[c, pl.ds(r0, ROWS_PER_TILE)])


def _scatter_body(src_hbm, dst_hbm, lins_hbm, zeros_hbm, out_hbm,
                  src_v, dst_v, b0, b1, acc, sem):
    c = lax.axis_index("c")
    s = lax.axis_index("s")
    w = c * NS + s
    n_seg = jnp.where(c == 0, NCH0 // SEGLEN, NCH1 // SEGLEN)
    r0 = s * ROWS_PER_TILE
    pltpu.sync_copy(zeros_hbm.at[pl.ds(r0, ROWS_PER_TILE)],
                    acc.at[pl.ds(r0, ROWS_PER_TILE)])
    plsc.subcore_barrier()
    bufs = [b0, b1]

    def seg_body(sg, carry):
        pltpu.sync_copy(src_hbm.at[w, pl.ds(sg * SEGLEN, SEGLEN)], src_v)
        pltpu.sync_copy(dst_hbm.at[w, pl.ds(sg * SEGLEN, SEGLEN)], dst_v)

        def grp(t, carry2):
            descs = []
            for b in range(NBUF):
                descs.append(pltpu.async_copy(
                    lins_hbm.at[src_v.at[NBUF * t + b]], bufs[b], sem))
            for d_ in descs:
                d_.wait()
            for b in range(NBUF):
                pltpu.sync_copy(bufs[b], acc.at[dst_v.at[NBUF * t + b]],
                                add=True)
            return carry2

        lax.fori_loop(0, SEGLEN // NBUF, grp, 0, unroll=False)
        return carry

    lax.fori_loop(0, n_seg, seg_body, 0, unroll=False)
    plsc.subcore_barrier()
    pltpu.sync_copy(acc.at[pl.ds(r0, ROWS_PER_TILE)],
                    out_hbm.at[c, pl.ds(r0, ROWS_PER_TILE)])


def _make_sc_kernels(interpret=False):
    deg = pl.kernel(
        _deg_body,
        out_type=jax.ShapeDtypeStruct((NC, N_ACC, 128), jnp.float32),
        mesh=_mesh,
        scratch_types=[
            pltpu.VMEM((NCH, CHUNK), jnp.int32),
            pltpu.VMEM((CHUNK, 128), jnp.float32),
            pltpu.VMEM_SHARED((N_ACC, 128), jnp.float32),
        ],
        interpret=interpret,
    )
    scat = pl.kernel(
        _scatter_body,
        out_type=jax.ShapeDtypeStruct((NC, N_ACC, D), jnp.float32),
        mesh=_mesh,
        scratch_types=(
            [pltpu.VMEM((SEGLEN, CHUNK), jnp.int32)] * 2
            + [pltpu.VMEM((CHUNK, D), jnp.float32)] * NBUF
            + [pltpu.VMEM_SHARED((N_ACC, D), jnp.float32)]
            + [pltpu.SemaphoreType.DMA]
        ),
        interpret=interpret,
    )
    return deg, scat


_deg_kernel, _scatter_kernel = _make_sc_kernels()


def _lins_body(x_ref, w_ref, d0_ref, d1_ref, o_ref):
    deg = d0_ref[:, 0:1] + d1_ref[:, 0:1] + 1.0
    dinv = lax.rsqrt(deg)
    lin = jnp.dot(x_ref[...], w_ref[...], preferred_element_type=jnp.float32)
    o_ref[...] = lin * dinv


def _lins_call(x, W, d0, d1):
    nb = 10
    rb = N_NODES // nb
    return pl.pallas_call(
        _lins_body,
        grid=(nb,),
        in_specs=[
            pl.BlockSpec((rb, D), lambda i: (i, 0)),
            pl.BlockSpec((D, D), lambda i: (0, 0)),
            pl.BlockSpec((rb, D), lambda i: (i, 0)),
            pl.BlockSpec((rb, D), lambda i: (i, 0)),
        ],
        out_specs=pl.BlockSpec((rb, D), lambda i: (i, 0)),
        out_shape=jax.ShapeDtypeStruct((N_NODES, D), jnp.float32),
    )(x, W, d0, d1)


def _out_body(p0_ref, p1_ref, lins_ref, d0_ref, d1_ref, b_ref, o_ref):
    deg = d0_ref[:, 0:1] + d1_ref[:, 0:1] + 1.0
    dinv = lax.rsqrt(deg)
    tot = (p0_ref[...] + p1_ref[...] + lins_ref[...]) * dinv + b_ref[...]
    o_ref[...] = jnp.maximum(tot, 0.0)


def _out_call(p0, p1, lins, d0, d1, b):
    nb = 10
    rb = N_NODES // nb
    return pl.pallas_call(
        _out_body,
        grid=(nb,),
        in_specs=[
            pl.BlockSpec((rb, D), lambda i: (i, 0)),
            pl.BlockSpec((rb, D), lambda i: (i, 0)),
            pl.BlockSpec((rb, D), lambda i: (i, 0)),
            pl.BlockSpec((rb, D), lambda i: (i, 0)),
            pl.BlockSpec((rb, D), lambda i: (i, 0)),
            pl.BlockSpec((1, D), lambda i: (0, 0)),
        ],
        out_specs=pl.BlockSpec((rb, D), lambda i: (i, 0)),
        out_shape=jax.ShapeDtypeStruct((N_NODES, D), jnp.float32),
    )(p0, p1, lins, d0, d1, b)


@jax.jit
def kernel(x, edge_index, W, b):
    src = edge_index[0].astype(jnp.int32)
    dst = edge_index[1].astype(jnp.int32)
    pad = E_PAD - N_EDGES
    src_p = jnp.concatenate([src, jnp.zeros((pad,), jnp.int32)])
    dst_p = jnp.concatenate([dst, jnp.full((pad,), TRASH, jnp.int32)])
    src_t = src_p.reshape(NW, NCH, CHUNK)
    dst_t = dst_p.reshape(NW, NCH, CHUNK)

    n0 = NS * NCH0 * CHUNK
    def _split(flat, fill):
        a0 = flat[:n0].reshape(NS, NCH0, CHUNK)
        a1 = flat[n0:].reshape(NS, NCH1, CHUNK)
        a0p = jnp.pad(a0, ((0, 0), (0, NCHMAX - NCH0), (0, 0)),
                      constant_values=fill)
        a1p = jnp.pad(a1, ((0, 0), (0, NCHMAX - NCH1), (0, 0)),
                      constant_values=fill)
        return jnp.concatenate([a0p, a1p], axis=0)
    src_u = _split(src_p, 0)
    dst_u = _split(dst_p, TRASH)

    onehot = jnp.zeros((CHUNK, 128), jnp.float32).at[:, 0].set(1.0)
    zeros16 = jnp.zeros((N_ACC, 128), jnp.float32)
    zerosD = jnp.zeros((N_ACC, D), jnp.float32)

    degacc = _deg_kernel(dst_t, onehot, zeros16)
    d0 = degacc[0]
    d1 = degacc[1]

    lins = _lins_call(x, W, d0[:N_NODES], d1[:N_NODES])

    parts = _scatter_kernel(src_u, dst_u, lins, zerosD)

    return _out_call(parts[0, :N_NODES], parts[1, :N_NODES], lins,
                     d0[:N_NODES], d1[:N_NODES], b.reshape(1, D))
